# baseline (speedup 1.0000x reference)
import numpy as np
import jax
import jax.numpy as jnp
from jax.experimental import pallas as pl

_N_LIST = [48, 192, 768, 3072, 12288, 49152]
_B = 4
_OFFS = (1, 2, 3, 4, -1, -2, -3, -4)

_INTERPRET = False


def _diag_coeffs(n, seed):
    rng = np.random.RandomState(seed)
    vals = rng.uniform(-0.05, 0.05, size=8 * n).astype(np.float32).reshape(n, 8)
    return np.stack([np.roll(vals[:, j], off) for j, off in enumerate(_OFFS)], axis=1)


_COEFFS = {n: _diag_coeffs(n, 100 + i) for i, n in enumerate(_N_LIST) if i >= 1}


def _cext_np(n, T):
    c = _COEFFS[n]
    nt = n // T
    idx = (np.arange(-8, T + 8)[None, :] + np.arange(nt)[:, None] * T) % n
    return c[idx]


def _halos(x, Tc, h):
    n, L = x.shape
    nt = n // Tc
    xr = x.reshape(nt, Tc, L)
    lh = jnp.roll(xr[:, Tc - h:], 1, axis=0)
    rh = jnp.roll(xr[:, :h], -1, axis=0)
    return lh, rh


def _fold(x):
    B, n, C = x.shape
    return x.transpose(1, 0, 2).reshape(n, B * C)


def _group_w(w, gsz, O):
    C = w.shape[1]
    Wg = jnp.zeros((gsz * w.shape[1], gsz * 3 * O), jnp.float32)
    for i in range(gsz):
        for k in range(3):
            Wg = Wg.at[i * C:(i + 1) * C,
                       k * gsz * O + i * O:k * gsz * O + (i + 1) * O].set(w[k])
    return Wg


def _make_conv(n, T, streams, O, with_stats):
    B = _B
    nt = n // T
    BO = B * O
    cext = _cext_np(n, T)
    gszs = [min(B, max(1, 256 // s['C'])) for s in streams]

    in_specs = [pl.BlockSpec((1, T + 16, 8), lambda t: (t, 0, 0))]
    for s in streams:
        C = s['C']
        u = 4 if s['unpool'] else 1
        Tc, h = T // u, 8 // u
        in_specs.append(pl.BlockSpec((Tc, B * C), lambda t: (t, 0)))
        in_specs.append(pl.BlockSpec((1, h, B * C), lambda t: (t, 0, 0)))
        in_specs.append(pl.BlockSpec((1, h, B * C), lambda t: (t, 0, 0)))
        if s['affine']:
            in_specs.append(pl.BlockSpec((1, B * C), lambda t: (0, 0)))
            in_specs.append(pl.BlockSpec((1, B * C), lambda t: (0, 0)))
    for s, gsz in zip(streams, gszs):
        in_specs.append(pl.BlockSpec((gsz * s['C'], gsz * 3 * O),
                                     lambda t: (0, 0)))
    in_specs.append(pl.BlockSpec((1, BO), lambda t: (0, 0)))

    out_specs = [pl.BlockSpec((T, BO), lambda t: (t, 0))]
    out_shape = [jax.ShapeDtypeStruct((n, BO), jnp.float32)]
    if with_stats:
        out_specs.append(pl.BlockSpec((8, BO), lambda t: (0, 0)))
        out_shape.append(jax.ShapeDtypeStruct((8, BO), jnp.float32))

    def body(*refs):
        refs = list(refs)
        cext_ref = refs.pop(0)
        stream_refs = []
        for s in streams:
            r = [refs.pop(0), refs.pop(0), refs.pop(0)]
            if s['affine']:
                r += [refs.pop(0), refs.pop(0)]
            stream_refs.append(r)
        w_refs = [refs.pop(0) for _ in streams]
        bias_ref = refs.pop(0)
        out_ref = refs.pop(0)
        stats_ref = refs.pop(0) if with_stats else None

        ce = cext_ref[0]
        u_acc = [None, None, None]
        for s, gsz, srefs, w_ref in zip(streams, gszs, stream_refs, w_refs):
            C = s['C']
            u = 4 if s['unpool'] else 1
            Tc, h = T // u, 8 // u
            x_ref, l_ref, r_ref = srefs[:3]
            xe = jnp.concatenate([l_ref[0], x_ref[...], r_ref[0]], axis=0)
            if s['affine']:
                xe = jnp.maximum(xe * srefs[3][...] + srefs[4][...], 0.0)
            rows = Tc + 2 * h
            parts = [[], [], []]
            for gi in range(B // gsz):
                xg = xe[:, gi * gsz * C:(gi + 1) * gsz * C]
                ug = jnp.dot(xg, w_ref[...], preferred_element_type=jnp.float32)
                for k in range(3):
                    parts[k].append(ug[:, k * gsz * O:(k + 1) * gsz * O])
            for k in range(3):
                m = parts[k][0] if len(parts[k]) == 1 else \
                    jnp.concatenate(parts[k], axis=1)
                if u == 4:
                    m = jnp.broadcast_to(m[:, None, :], (rows, 4, BO))
                    m = m.reshape(T + 16, BO)
                u_acc[k] = m if u_acc[k] is None else u_acc[k] + m
        u0, u1, u2 = u_acc
        v = 0.5 * u2[4:T + 12]
        for j, off in enumerate(_OFFS):
            v = v + ce[4:T + 12, j:j + 1] * u2[4 - off:T + 12 - off]
        sarr = u1[4:T + 12] + 2.0 * v
        w = 0.5 * sarr[4:T + 4]
        for j, off in enumerate(_OFFS):
            w = w + ce[8:T + 8, j:j + 1] * sarr[4 - off:T + 4 - off]
        y = u0[8:T + 8] - u2[8:T + 8] + w + bias_ref[...]
        out_ref[...] = y
        if with_stats:
            t = pl.program_id(0)
            upd = jnp.concatenate([
                jnp.sum(y, axis=0, keepdims=True),
                jnp.sum(y * y, axis=0, keepdims=True),
                jnp.zeros((6, BO), jnp.float32),
            ], axis=0)

            @pl.when(t == 0)
            def _init():
                stats_ref[...] = jnp.zeros((8, BO), jnp.float32)

            stats_ref[...] = stats_ref[...] + upd

    def call(stream_args, w_list, bias):
        args = [jnp.asarray(cext)]
        for s, (x, ac) in zip(streams, stream_args):
            u = 4 if s['unpool'] else 1
            Tc, h = T // u, 8 // u
            lh, rh = _halos(x, Tc, h)
            args += [x, lh, rh]
            if s['affine']:
                args += [jnp.tile(ac[0], B).reshape(1, -1),
                         jnp.tile(ac[1], B).reshape(1, -1)]
        for w, gsz in zip(w_list, gszs):
            args.append(_group_w(w, gsz, O))
        args.append(jnp.tile(bias, B).reshape(1, -1))
        return pl.pallas_call(
            body,
            grid=(nt,),
            in_specs=in_specs,
            out_specs=out_specs,
            out_shape=out_shape,
            interpret=_INTERPRET,
        )(*args)

    return call


_CFGS = [
    (192, 192, [dict(C=512, unpool=True, affine=False)], 512, True),
    (192, 192, [dict(C=512, unpool=False, affine=True),
                dict(C=512, unpool=False, affine=False)], 512, True),
    (768, 768, [dict(C=512, unpool=True, affine=True)], 256, True),
    (768, 768, [dict(C=256, unpool=False, affine=True),
                dict(C=512, unpool=False, affine=False)], 256, True),
    (3072, 768, [dict(C=256, unpool=True, affine=True)], 128, True),
    (3072, 768, [dict(C=128, unpool=False, affine=True),
                 dict(C=256, unpool=False, affine=False)], 128, True),
    (12288, 1024, [dict(C=128, unpool=True, affine=True)], 64, True),
    (12288, 1024, [dict(C=64, unpool=False, affine=True),
                   dict(C=128, unpool=False, affine=False)], 64, True),
    (49152, 2048, [dict(C=64, unpool=True, affine=True)], 32, True),
    (49152, 2048, [dict(C=32, unpool=False, affine=True)], 8, False),
]

_CONVS = [_make_conv(*cfg) for cfg in _CFGS]


def _bn_affine(stats, n, g, be):
    cnt = float(_B * n)
    sums = stats[:2].reshape(2, _B, -1).sum(axis=1)
    m = sums[0] / cnt
    v = sums[1] / cnt - m * m
    a = g * jax.lax.rsqrt(v + 1e-5)
    c = be - m * a
    return a, c


_ABLATE = 0


def kernel(x_enc0, x_enc1, x_enc2, x_enc3, x_enc4, params):
    p = params
    encs = [_fold(x) for x in (x_enc1, x_enc2, x_enc3, x_enc4)]

    y, st = _CONVS[0]([(_fold(x_enc0), None)], [p['l1_pool_w']], p['l1_pool_b'])
    ac = _bn_affine(st, 192, p['l1_pool_g'], p['l1_pool_be'])

    names = ['l1', 'l2', 'l3', 'l4']
    ci = 1
    for i, nm in enumerate(names):
        w = p[nm + '_w']
        C1 = _CFGS[ci][2][0]['C']
        y, st = _CONVS[ci]([(y, ac), (encs[i], None)],
                           [w[:, :C1], w[:, C1:]], p[nm + '_b'])
        ac = _bn_affine(st, _CFGS[ci][0], p[nm + '_g'], p[nm + '_be'])
        ci += 1
        if _ABLATE and ci > _ABLATE:
            return jnp.zeros((4, 49152, 3), jnp.float32) + jnp.sum(y)
        if nm != 'l4':
            pw = p['l%d_pool_w' % (i + 2)]
            y, st = _CONVS[ci]([(y, ac)], [pw], p['l%d_pool_b' % (i + 2)])
            ac = _bn_affine(st, _CFGS[ci][0],
                            p['l%d_pool_g' % (i + 2)], p['l%d_pool_be' % (i + 2)])
            ci += 1

    y, st = _CONVS[8]([(y, ac)], [p['l5_pool_w']], p['l5_pool_b'])
    ac = _bn_affine(st, 49152, p['l5_pool_g'], p['l5_pool_be'])
    w5 = jnp.pad(p['l5_w'], ((0, 0), (0, 0), (0, 5)))
    b5 = jnp.pad(p['l5_b'], (0, 5))
    (y,) = _CONVS[9]([(y, ac)], [w5], b5)
    return y.reshape(49152, 4, 8).transpose(1, 0, 2)[:, :, :3]


# speedup vs baseline: 1.1414x; 1.1414x over previous
import numpy as np
import jax
import jax.numpy as jnp
from jax.experimental import pallas as pl
from jax.sharding import PartitionSpec as _P

_N_LIST = [48, 192, 768, 3072, 12288, 49152]
_B = 4
_OFFS = (1, 2, 3, 4, -1, -2, -3, -4)

_NSH = 1

_INTERPRET = False


def _diag_coeffs(n, seed):
    rng = np.random.RandomState(seed)
    vals = rng.uniform(-0.05, 0.05, size=8 * n).astype(np.float32).reshape(n, 8)
    return np.stack([np.roll(vals[:, j], off) for j, off in enumerate(_OFFS)], axis=1)


_COEFFS = {n: _diag_coeffs(n, 100 + i) for i, n in enumerate(_N_LIST) if i >= 1}


def _cext_np(n, T):
    c = _COEFFS[n]
    nt = n // T
    idx = (np.arange(-8, T + 8)[None, :] + np.arange(nt)[:, None] * T) % n
    return c[idx]


def _fold(x):
    B, n, C = x.shape
    return x.transpose(1, 0, 2).reshape(n, B * C)


def _group_w(w, gsz, O):
    C = w.shape[1]
    Wg = jnp.zeros((gsz * C, gsz * 3 * O), jnp.float32)
    for i in range(gsz):
        for k in range(3):
            Wg = Wg.at[i * C:(i + 1) * C,
                       k * gsz * O + i * O:k * gsz * O + (i + 1) * O].set(w[k])
    return Wg


def _tree_sum(ts):
    while len(ts) > 1:
        ts = [a + b for a, b in zip(ts[::2], ts[1::2])] + \
            (ts[-1:] if len(ts) % 2 else [])
    return ts[0]


def _make_conv(n, T, streams, O, with_stats, rows_mode):
    B = _B
    nl = n // _NSH
    ntb = nl // T
    nt = ntb * B if rows_mode else ntb
    BO = O if rows_mode else B * O
    lanes_in = (lambda C: C) if rows_mode else (lambda C: B * C)
    gszs = [1 if rows_mode else min(B, max(1, 256 // s['C'])) for s in streams]

    in_specs = [pl.BlockSpec((1, T + 16, 8), lambda t: (t, 0, 0))]
    for s in streams:
        Li = lanes_in(s['C'])
        u = 4 if s['unpool'] else 1
        Tc, h = T // u, 8 // u
        in_specs.append(pl.BlockSpec((Tc, Li), lambda t: (t, 0)))
        in_specs.append(pl.BlockSpec((1, h, Li), lambda t: (t, 0, 0)))
        in_specs.append(pl.BlockSpec((1, h, Li), lambda t: (t, 0, 0)))
        if s['affine']:
            in_specs.append(pl.BlockSpec((1, Li), lambda t: (0, 0)))
            in_specs.append(pl.BlockSpec((1, Li), lambda t: (0, 0)))
    for s, gsz in zip(streams, gszs):
        in_specs.append(pl.BlockSpec((gsz * s['C'], gsz * 3 * O),
                                     lambda t: (0, 0)))
    in_specs.append(pl.BlockSpec((1, BO), lambda t: (0, 0)))

    rtot = nl * B if rows_mode else nl
    out_specs = [pl.BlockSpec((T, BO), lambda t: (t, 0))]
    out_shape = [jax.ShapeDtypeStruct((rtot, BO), jnp.float32)]
    if with_stats:
        out_specs.append(pl.BlockSpec((8, BO), lambda t: (0, 0)))
        out_shape.append(jax.ShapeDtypeStruct((8, BO), jnp.float32))

    def body(*refs):
        refs = list(refs)
        cext_ref = refs.pop(0)
        stream_refs = []
        for s in streams:
            r = [refs.pop(0), refs.pop(0), refs.pop(0)]
            if s['affine']:
                r += [refs.pop(0), refs.pop(0)]
            stream_refs.append(r)
        w_refs = [refs.pop(0) for _ in streams]
        bias_ref = refs.pop(0)
        out_ref = refs.pop(0)
        stats_ref = refs.pop(0) if with_stats else None

        ce = cext_ref[0]
        u_acc = [None, None, None]
        for s, gsz, srefs, w_ref in zip(streams, gszs, stream_refs, w_refs):
            C = s['C']
            u = 4 if s['unpool'] else 1
            Tc, h = T // u, 8 // u
            x_ref, l_ref, r_ref = srefs[:3]
            xe = jnp.concatenate([l_ref[0], x_ref[...], r_ref[0]], axis=0)
            if s['affine']:
                xe = jnp.maximum(xe * srefs[3][...] + srefs[4][...], 0.0)
            rows = Tc + 2 * h
            ngr = (1 if rows_mode else B) // gsz
            parts = [[], [], []]
            for gi in range(ngr):
                xg = xe if ngr == 1 else \
                    xe[:, gi * gsz * C:(gi + 1) * gsz * C]
                ug = jnp.dot(xg, w_ref[...], preferred_element_type=jnp.float32)
                for k in range(3):
                    parts[k].append(ug[:, k * gsz * O:(k + 1) * gsz * O])
            for k in range(3):
                m = parts[k][0] if len(parts[k]) == 1 else \
                    jnp.concatenate(parts[k], axis=1)
                if u == 4:
                    m = jnp.broadcast_to(m[:, None, :], (rows, 4, BO))
                    m = m.reshape(T + 16, BO)
                u_acc[k] = m if u_acc[k] is None else u_acc[k] + m
        u0, u1, u2 = u_acc
        v = _tree_sum([0.5 * u2[4:T + 12]] + [
            ce[4:T + 12, j:j + 1] * u2[4 - off:T + 12 - off]
            for j, off in enumerate(_OFFS)])
        sarr = u1[4:T + 12] + 2.0 * v
        w = _tree_sum([0.5 * sarr[4:T + 4]] + [
            ce[8:T + 8, j:j + 1] * sarr[4 - off:T + 4 - off]
            for j, off in enumerate(_OFFS)])
        y = u0[8:T + 8] - u2[8:T + 8] + w + bias_ref[...]
        out_ref[...] = y
        if with_stats:
            t = pl.program_id(0)
            upd = jnp.concatenate([
                jnp.sum(y, axis=0, keepdims=True),
                jnp.sum(y * y, axis=0, keepdims=True),
                jnp.zeros((6, BO), jnp.float32),
            ], axis=0)

            @pl.when(t == 0)
            def _init():
                stats_ref[...] = jnp.zeros((8, BO), jnp.float32)

            stats_ref[...] = stats_ref[...] + upd

    def halos(x, Tc, h):
        L = x.shape[-1]
        if rows_mode:
            xr = x.reshape(B, -1, Tc, L)
            heads, tails = xr[:, :, :h], xr[:, :, Tc - h:]
            lh = jnp.roll(tails, 1, axis=1).reshape(-1, h, L)
            rh = jnp.roll(heads, -1, axis=1).reshape(-1, h, L)
            return lh, rh
        ntl = x.shape[0] // Tc
        xr = x.reshape(ntl, Tc, L)
        heads, tails = xr[:, :h], xr[:, Tc - h:]
        prev_tail = jax.lax.ppermute(
            tails[-1:], 'd', [(i, (i + 1) % _NSH) for i in range(_NSH)])
        next_head = jax.lax.ppermute(
            heads[:1], 'd', [(i, (i - 1) % _NSH) for i in range(_NSH)])
        lh = jnp.concatenate([prev_tail, tails[:-1]], axis=0)
        rh = jnp.concatenate([heads[1:], next_head], axis=0)
        return lh, rh

    def call(cext_loc, stream_args, w_list, bias):
        args = [cext_loc]
        for s, (x, ac) in zip(streams, stream_args):
            u = 4 if s['unpool'] else 1
            Tc, h = T // u, 8 // u
            lh, rh = halos(x, Tc, h)
            args += [x, lh, rh]
            if s['affine']:
                tile = (lambda z: z) if rows_mode else (lambda z: jnp.tile(z, B))
                args += [tile(ac[0]).reshape(1, -1), tile(ac[1]).reshape(1, -1)]
        for w, gsz in zip(w_list, gszs):
            if rows_mode:
                args.append(jnp.concatenate([w[0], w[1], w[2]], axis=1))
            else:
                args.append(_group_w(w, gsz, O))
        args.append((bias if rows_mode else jnp.tile(bias, B)).reshape(1, -1))
        return pl.pallas_call(
            body,
            grid=(nt,),
            in_specs=in_specs,
            out_specs=out_specs,
            out_shape=out_shape,
            interpret=_INTERPRET,
        )(*args)

    return call


_CFGS = [
    (192, 192, [dict(C=512, unpool=True, affine=False)], 512, True, True),
    (192, 192, [dict(C=512, unpool=False, affine=True),
                dict(C=512, unpool=False, affine=False)], 512, True, True),
    (768, 768, [dict(C=512, unpool=True, affine=True)], 256, True, True),
    (768, 768, [dict(C=256, unpool=False, affine=True),
                dict(C=512, unpool=False, affine=False)], 256, True, True),
    (3072, 768, [dict(C=256, unpool=True, affine=True)], 128, True, True),
    (3072, 768, [dict(C=128, unpool=False, affine=True),
                 dict(C=256, unpool=False, affine=False)], 128, True, True),
    (12288, 1024, [dict(C=128, unpool=True, affine=True)], 64, True, False),
    (12288, 1024, [dict(C=64, unpool=False, affine=True),
                   dict(C=128, unpool=False, affine=False)], 64, True, False),
    (49152, 4096, [dict(C=64, unpool=True, affine=True)], 32, True, False),
    (49152, 4096, [dict(C=32, unpool=False, affine=True)], 8, False, False),
]

_CONVS = [_make_conv(*cfg) for cfg in _CFGS]


def _cext_for(ci):
    n, T = _CFGS[ci][0], _CFGS[ci][1]
    base = _cext_np(n, T)
    if _CFGS[ci][5]:
        base = np.tile(base, (_B, 1, 1))
    return base


_CEXTS = [_cext_for(i) for i in range(len(_CFGS))]


def _bn_affine(stats, n, g, be, rows_mode):
    cnt = float(_B * n)
    s2 = jax.lax.psum(stats[:2], 'd')
    sums = s2 if rows_mode else s2.reshape(2, _B, -1).sum(axis=1)
    m = sums[0] / cnt
    v = sums[1] / cnt - m * m
    a = g * jax.lax.rsqrt(v + 1e-5)
    c = be - m * a
    return a, c


def _pipeline(x_enc0, x_enc1, x_enc2, x_enc3, x_enc4, params, *cexts):
    p = params
    encs = [x.reshape(-1, x.shape[-1]) for x in (x_enc1, x_enc2, x_enc3)]
    encs.append(_fold(x_enc4))

    y, st = _CONVS[0](cexts[0], [(x_enc0.reshape(-1, 512), None)],
                      [p['l1_pool_w']], p['l1_pool_b'])
    ac = _bn_affine(st, 192, p['l1_pool_g'], p['l1_pool_be'], True)

    names = ['l1', 'l2', 'l3', 'l4']
    ci = 1
    for i, nm in enumerate(names):
        w = p[nm + '_w']
        C1 = _CFGS[ci][2][0]['C']
        y, st = _CONVS[ci](cexts[ci], [(y, ac), (encs[i], None)],
                           [w[:, :C1], w[:, C1:]], p[nm + '_b'])
        ac = _bn_affine(st, _CFGS[ci][0], p[nm + '_g'], p[nm + '_be'],
                        _CFGS[ci][5])
        ci += 1
        if nm == 'l3':
            y = _fold(y.reshape(_B, 3072, 128))
        if nm != 'l4':
            pw = p['l%d_pool_w' % (i + 2)]
            y, st = _CONVS[ci](cexts[ci], [(y, ac)], [pw],
                               p['l%d_pool_b' % (i + 2)])
            ac = _bn_affine(st, _CFGS[ci][0], p['l%d_pool_g' % (i + 2)],
                            p['l%d_pool_be' % (i + 2)], _CFGS[ci][5])
            ci += 1

    y, st = _CONVS[8](cexts[8], [(y, ac)], [p['l5_pool_w']], p['l5_pool_b'])
    ac = _bn_affine(st, 49152, p['l5_pool_g'], p['l5_pool_be'], False)
    w5 = jnp.pad(p['l5_w'], ((0, 0), (0, 0), (0, 5)))
    b5 = jnp.pad(p['l5_b'], (0, 5))
    (y,) = _CONVS[9](cexts[9], [(y, ac)], [w5], b5)
    nl = 49152 // _NSH
    return y.reshape(nl, 4, 8).transpose(1, 0, 2)[:, :, :3]


def kernel(x_enc0, x_enc1, x_enc2, x_enc3, x_enc4, params):
    mesh = jax.make_mesh((_NSH,), ('d',))
    enc = _P(None, 'd', None)
    f = jax.shard_map(
        _pipeline, mesh=mesh,
        in_specs=(enc, enc, enc, enc, enc, _P()) + (_P('d', None, None),) * 10,
        out_specs=_P(None, 'd', None),
        check_vma=False)
    from jax.sharding import NamedSharding as _NS
    encs = [jax.reshard(x, _NS(mesh, enc))
            for x in (x_enc0, x_enc1, x_enc2, x_enc3, x_enc4)]
    params = jax.tree.map(lambda x: jax.reshard(x, _NS(mesh, _P())), params)
    cexts = [jax.reshard(jnp.asarray(c), _NS(mesh, _P('d', None, None)))
             for c in _CEXTS]
    return f(*encs, params, *cexts)


# speedup vs baseline: 1.1416x; 1.0002x over previous
import numpy as np
import jax
import jax.numpy as jnp
from jax.experimental import pallas as pl
from jax.sharding import PartitionSpec as _P

_N_LIST = [48, 192, 768, 3072, 12288, 49152]
_B = 4
_OFFS = (1, 2, 3, 4, -1, -2, -3, -4)

_NSH = 1

_INTERPRET = False


def _diag_coeffs(n, seed):
    rng = np.random.RandomState(seed)
    vals = rng.uniform(-0.05, 0.05, size=8 * n).astype(np.float32).reshape(n, 8)
    return np.stack([np.roll(vals[:, j], off) for j, off in enumerate(_OFFS)], axis=1)


_COEFFS = {n: _diag_coeffs(n, 100 + i) for i, n in enumerate(_N_LIST) if i >= 1}


def _cext_np(n, T):
    c = _COEFFS[n]
    nt = n // T
    idx = (np.arange(-8, T + 8)[None, :] + np.arange(nt)[:, None] * T) % n
    return c[idx]


def _fold(x):
    B, n, C = x.shape
    return x.transpose(1, 0, 2).reshape(n, B * C)


def _group_w(w, gsz, O):
    C = w.shape[1]
    Wg = jnp.zeros((gsz * C, gsz * 3 * O), jnp.float32)
    for i in range(gsz):
        for k in range(3):
            Wg = Wg.at[i * C:(i + 1) * C,
                       k * gsz * O + i * O:k * gsz * O + (i + 1) * O].set(w[k])
    return Wg


def _tree_sum(ts):
    while len(ts) > 1:
        ts = [a + b for a, b in zip(ts[::2], ts[1::2])] + \
            (ts[-1:] if len(ts) % 2 else [])
    return ts[0]


def _make_conv(n, T, streams, O, with_stats, rows_mode):
    B = _B
    nl = n // _NSH
    ntb = nl // T
    nt = ntb * B if rows_mode else ntb
    BO = O if rows_mode else B * O
    lanes_in = (lambda C: C) if rows_mode else (lambda C: B * C)
    gszs = [1 if rows_mode else min(B, max(1, 256 // s['C'])) for s in streams]

    in_specs = [pl.BlockSpec((1, T + 16, 8), lambda t: (t, 0, 0))]
    for s in streams:
        Li = lanes_in(s['C'])
        u = 4 if s['unpool'] else 1
        Tc, h = T // u, 8 // u
        in_specs.append(pl.BlockSpec((Tc, Li), lambda t: (t, 0)))
        in_specs.append(pl.BlockSpec((1, h, Li), lambda t: (t, 0, 0)))
        in_specs.append(pl.BlockSpec((1, h, Li), lambda t: (t, 0, 0)))
        if s['affine']:
            in_specs.append(pl.BlockSpec((1, Li), lambda t: (0, 0)))
            in_specs.append(pl.BlockSpec((1, Li), lambda t: (0, 0)))
    for s, gsz in zip(streams, gszs):
        in_specs.append(pl.BlockSpec((gsz * s['C'], gsz * 3 * O),
                                     lambda t: (0, 0)))
    in_specs.append(pl.BlockSpec((1, BO), lambda t: (0, 0)))

    rtot = nl * B if rows_mode else nl
    out_specs = [pl.BlockSpec((T, BO), lambda t: (t, 0))]
    out_shape = [jax.ShapeDtypeStruct((rtot, BO), jnp.float32)]
    if with_stats:
        out_specs.append(pl.BlockSpec((8, BO), lambda t: (0, 0)))
        out_shape.append(jax.ShapeDtypeStruct((8, BO), jnp.float32))

    def body(*refs):
        refs = list(refs)
        cext_ref = refs.pop(0)
        stream_refs = []
        for s in streams:
            r = [refs.pop(0), refs.pop(0), refs.pop(0)]
            if s['affine']:
                r += [refs.pop(0), refs.pop(0)]
            stream_refs.append(r)
        w_refs = [refs.pop(0) for _ in streams]
        bias_ref = refs.pop(0)
        out_ref = refs.pop(0)
        stats_ref = refs.pop(0) if with_stats else None

        ce = cext_ref[0]
        u_acc = [None, None, None]
        for s, gsz, srefs, w_ref in zip(streams, gszs, stream_refs, w_refs):
            C = s['C']
            u = 4 if s['unpool'] else 1
            Tc, h = T // u, 8 // u
            x_ref, l_ref, r_ref = srefs[:3]
            xe = jnp.concatenate([l_ref[0], x_ref[...], r_ref[0]], axis=0)
            if s['affine']:
                xe = jnp.maximum(xe * srefs[3][...] + srefs[4][...], 0.0)
            rows = Tc + 2 * h
            ngr = (1 if rows_mode else B) // gsz
            parts = [[], [], []]
            for gi in range(ngr):
                xg = xe if ngr == 1 else \
                    xe[:, gi * gsz * C:(gi + 1) * gsz * C]
                ug = jnp.dot(xg, w_ref[...], preferred_element_type=jnp.float32)
                for k in range(3):
                    parts[k].append(ug[:, k * gsz * O:(k + 1) * gsz * O])
            for k in range(3):
                m = parts[k][0] if len(parts[k]) == 1 else \
                    jnp.concatenate(parts[k], axis=1)
                if u == 4:
                    m = jnp.broadcast_to(m[:, None, :], (rows, 4, BO))
                    m = m.reshape(T + 16, BO)
                u_acc[k] = m if u_acc[k] is None else u_acc[k] + m
        u0, u1, u2 = u_acc
        v = _tree_sum([0.5 * u2[4:T + 12]] + [
            ce[4:T + 12, j:j + 1] * u2[4 - off:T + 12 - off]
            for j, off in enumerate(_OFFS)])
        sarr = u1[4:T + 12] + 2.0 * v
        w = _tree_sum([0.5 * sarr[4:T + 4]] + [
            ce[8:T + 8, j:j + 1] * sarr[4 - off:T + 4 - off]
            for j, off in enumerate(_OFFS)])
        y = u0[8:T + 8] - u2[8:T + 8] + w + bias_ref[...]
        out_ref[...] = y
        if with_stats:
            t = pl.program_id(0)
            upd = jnp.concatenate([
                jnp.sum(y, axis=0, keepdims=True),
                jnp.sum(y * y, axis=0, keepdims=True),
                jnp.zeros((6, BO), jnp.float32),
            ], axis=0)

            @pl.when(t == 0)
            def _init():
                stats_ref[...] = jnp.zeros((8, BO), jnp.float32)

            stats_ref[...] = stats_ref[...] + upd

    def halos(x, Tc, h):
        L = x.shape[-1]
        if rows_mode:
            xr = x.reshape(B, -1, Tc, L)
            heads, tails = xr[:, :, :h], xr[:, :, Tc - h:]
            lh = jnp.roll(tails, 1, axis=1).reshape(-1, h, L)
            rh = jnp.roll(heads, -1, axis=1).reshape(-1, h, L)
            return lh, rh
        ntl = x.shape[0] // Tc
        xr = x.reshape(ntl, Tc, L)
        heads, tails = xr[:, :h], xr[:, Tc - h:]
        prev_tail = jax.lax.ppermute(
            tails[-1:], 'd', [(i, (i + 1) % _NSH) for i in range(_NSH)])
        next_head = jax.lax.ppermute(
            heads[:1], 'd', [(i, (i - 1) % _NSH) for i in range(_NSH)])
        lh = jnp.concatenate([prev_tail, tails[:-1]], axis=0)
        rh = jnp.concatenate([heads[1:], next_head], axis=0)
        return lh, rh

    def call(cext_loc, stream_args, w_list, bias):
        args = [cext_loc]
        for s, (x, ac) in zip(streams, stream_args):
            u = 4 if s['unpool'] else 1
            Tc, h = T // u, 8 // u
            lh, rh = halos(x, Tc, h)
            args += [x, lh, rh]
            if s['affine']:
                tile = (lambda z: z) if rows_mode else (lambda z: jnp.tile(z, B))
                args += [tile(ac[0]).reshape(1, -1), tile(ac[1]).reshape(1, -1)]
        for w, gsz in zip(w_list, gszs):
            if rows_mode:
                args.append(jnp.concatenate([w[0], w[1], w[2]], axis=1))
            else:
                args.append(_group_w(w, gsz, O))
        args.append((bias if rows_mode else jnp.tile(bias, B)).reshape(1, -1))
        return pl.pallas_call(
            body,
            grid=(nt,),
            in_specs=in_specs,
            out_specs=out_specs,
            out_shape=out_shape,
            interpret=_INTERPRET,
        )(*args)

    return call


_CFGS = [
    (192, 192, [dict(C=512, unpool=True, affine=False)], 512, True, True),
    (192, 192, [dict(C=512, unpool=False, affine=True),
                dict(C=512, unpool=False, affine=False)], 512, True, True),
    (768, 768, [dict(C=512, unpool=True, affine=True)], 256, True, True),
    (768, 768, [dict(C=256, unpool=False, affine=True),
                dict(C=512, unpool=False, affine=False)], 256, True, True),
    (3072, 768, [dict(C=256, unpool=True, affine=True)], 128, True, True),
    (3072, 768, [dict(C=128, unpool=False, affine=True),
                 dict(C=256, unpool=False, affine=False)], 128, True, True),
    (12288, 2048, [dict(C=128, unpool=True, affine=True)], 64, True, False),
    (12288, 2048, [dict(C=64, unpool=False, affine=True),
                   dict(C=128, unpool=False, affine=False)], 64, True, False),
    (49152, 4096, [dict(C=64, unpool=True, affine=True)], 32, True, False),
    (49152, 4096, [dict(C=32, unpool=False, affine=True)], 8, False, False),
]

_CONVS = [_make_conv(*cfg) for cfg in _CFGS]


def _cext_for(ci):
    n, T = _CFGS[ci][0], _CFGS[ci][1]
    base = _cext_np(n, T)
    if _CFGS[ci][5]:
        base = np.tile(base, (_B, 1, 1))
    return base


_CEXTS = [_cext_for(i) for i in range(len(_CFGS))]


def _bn_affine(stats, n, g, be, rows_mode):
    cnt = float(_B * n)
    s2 = jax.lax.psum(stats[:2], 'd')
    sums = s2 if rows_mode else s2.reshape(2, _B, -1).sum(axis=1)
    m = sums[0] / cnt
    v = sums[1] / cnt - m * m
    a = g * jax.lax.rsqrt(v + 1e-5)
    c = be - m * a
    return a, c


def _pipeline(x_enc0, x_enc1, x_enc2, x_enc3, x_enc4, params, *cexts):
    p = params
    encs = [x.reshape(-1, x.shape[-1]) for x in (x_enc1, x_enc2, x_enc3)]
    encs.append(_fold(x_enc4))

    y, st = _CONVS[0](cexts[0], [(x_enc0.reshape(-1, 512), None)],
                      [p['l1_pool_w']], p['l1_pool_b'])
    ac = _bn_affine(st, 192, p['l1_pool_g'], p['l1_pool_be'], True)

    names = ['l1', 'l2', 'l3', 'l4']
    ci = 1
    for i, nm in enumerate(names):
        w = p[nm + '_w']
        C1 = _CFGS[ci][2][0]['C']
        y, st = _CONVS[ci](cexts[ci], [(y, ac), (encs[i], None)],
                           [w[:, :C1], w[:, C1:]], p[nm + '_b'])
        ac = _bn_affine(st, _CFGS[ci][0], p[nm + '_g'], p[nm + '_be'],
                        _CFGS[ci][5])
        ci += 1
        if nm == 'l3':
            y = _fold(y.reshape(_B, 3072, 128))
        if nm != 'l4':
            pw = p['l%d_pool_w' % (i + 2)]
            y, st = _CONVS[ci](cexts[ci], [(y, ac)], [pw],
                               p['l%d_pool_b' % (i + 2)])
            ac = _bn_affine(st, _CFGS[ci][0], p['l%d_pool_g' % (i + 2)],
                            p['l%d_pool_be' % (i + 2)], _CFGS[ci][5])
            ci += 1

    y, st = _CONVS[8](cexts[8], [(y, ac)], [p['l5_pool_w']], p['l5_pool_b'])
    ac = _bn_affine(st, 49152, p['l5_pool_g'], p['l5_pool_be'], False)
    w5 = jnp.pad(p['l5_w'], ((0, 0), (0, 0), (0, 5)))
    b5 = jnp.pad(p['l5_b'], (0, 5))
    (y,) = _CONVS[9](cexts[9], [(y, ac)], [w5], b5)
    nl = 49152 // _NSH
    return y.reshape(nl, 4, 8).transpose(1, 0, 2)[:, :, :3]


def kernel(x_enc0, x_enc1, x_enc2, x_enc3, x_enc4, params):
    mesh = jax.make_mesh((_NSH,), ('d',))
    enc = _P(None, 'd', None)
    f = jax.shard_map(
        _pipeline, mesh=mesh,
        in_specs=(enc, enc, enc, enc, enc, _P()) + (_P('d', None, None),) * 10,
        out_specs=_P(None, 'd', None),
        check_vma=False)
    from jax.sharding import NamedSharding as _NS
    encs = [jax.reshard(x, _NS(mesh, enc))
            for x in (x_enc0, x_enc1, x_enc2, x_enc3, x_enc4)]
    params = jax.tree.map(lambda x: jax.reshard(x, _NS(mesh, _P())), params)
    cexts = [jax.reshard(jnp.asarray(c), _NS(mesh, _P('d', None, None)))
             for c in _CEXTS]
    return f(*encs, params, *cexts)
